# baseline (speedup 1.0000x reference)
import functools

import jax
import jax.numpy as jnp
from jax import lax
from jax.experimental import pallas as pl
from jax.experimental.pallas import tpu as pltpu
from jax.experimental.pallas import tpu_sc as plsc

N_NODES = 10000
NUM_USERS = 4000
D = 128
E = 320000
N_LAYERS = 3

NC = 2
NS = 16
NW = NC * NS
CH = 128
TOT_CHK = 2560
NA = 57
NB = 103
MAXN = max(NA, NB)
DEG_CPT = 160
E_PAD = TOT_CHK * CH
NPAD = 10240
PAD_NODE = NPAD - 1
ROWS_PER_TILE = NPAD // NS

_mesh = plsc.VectorSubcoreMesh(core_axis_name="c", subcore_axis_name="s")


@functools.partial(
    pl.kernel,
    out_type=jax.ShapeDtypeStruct((NC * NPAD, D), jnp.float32),
    mesh=_mesh,
    scratch_types=[
        pltpu.VMEM((DEG_CPT, CH), jnp.int32),
        pltpu.VMEM((CH, D), jnp.float32),
        pltpu.VMEM_SHARED((NPAD, D), jnp.float32),
    ],
)
def _sc_degrees(idx_hbm, ones_hbm, zD_hbm,
                deg_out,
                idx_v, ones_v, hist_sp):
    c = lax.axis_index("c").astype(jnp.int32)
    s = lax.axis_index("s").astype(jnp.int32)
    row0 = s * jnp.int32(ROWS_PER_TILE)
    pltpu.sync_copy(zD_hbm.at[pl.ds(row0, ROWS_PER_TILE)],
                    hist_sp.at[pl.ds(row0, ROWS_PER_TILE)])
    pltpu.sync_copy(ones_hbm, ones_v)
    pltpu.sync_copy(idx_hbm.at[c, pl.ds(s * jnp.int32(DEG_CPT), DEG_CPT)],
                    idx_v)
    plsc.subcore_barrier()

    @pl.loop(jnp.int32(0), jnp.int32(DEG_CPT), step=jnp.int32(1))
    def _chunk(j):
        pltpu.sync_copy(ones_v, hist_sp.at[idx_v.at[j]], add=True)

    plsc.subcore_barrier()
    out0 = c * jnp.int32(NPAD) + row0
    pltpu.sync_copy(hist_sp.at[pl.ds(row0, ROWS_PER_TILE)],
                    deg_out.at[pl.ds(out0, ROWS_PER_TILE)])


@functools.partial(
    pl.kernel,
    out_type=jax.ShapeDtypeStruct((NC * NPAD, D), jnp.float32),
    mesh=_mesh,
    scratch_types=[
        pltpu.VMEM((MAXN, CH), jnp.int32),
        pltpu.VMEM((MAXN, CH), jnp.int32),
        pltpu.VMEM((CH, D), jnp.float32),
        pltpu.VMEM_SHARED((NPAD, D), jnp.float32),
        pltpu.SemaphoreType.DMA,
    ],
)
def _sc_scatter(g_hbm, src_hbm, dst_hbm, zD_hbm,
                acc_out,
                idx_s, idx_d, rows_v, acc_sp, sem):
    c = lax.axis_index("c").astype(jnp.int32)
    s = lax.axis_index("s").astype(jnp.int32)
    wid = c * jnp.int32(NS) + s
    row0 = s * jnp.int32(ROWS_PER_TILE)
    pltpu.sync_copy(zD_hbm.at[pl.ds(row0, ROWS_PER_TILE)],
                    acc_sp.at[pl.ds(row0, ROWS_PER_TILE)])
    pltpu.sync_copy(src_hbm.at[wid], idx_s)
    pltpu.sync_copy(dst_hbm.at[wid], idx_d)
    plsc.subcore_barrier()

    def _chunk(j):
        pltpu.async_copy(g_hbm.at[idx_s.at[j]], rows_v, sem).wait()
        pltpu.sync_copy(rows_v, acc_sp.at[idx_d.at[j]], add=True)

    @pl.when(c == jnp.int32(0))
    def _core_a():
        pl.loop(jnp.int32(0), jnp.int32(NA), step=jnp.int32(1))(_chunk)

    @pl.when(c != jnp.int32(0))
    def _core_b():
        pl.loop(jnp.int32(0), jnp.int32(NB), step=jnp.int32(1))(_chunk)

    plsc.subcore_barrier()
    out0 = c * jnp.int32(NPAD) + row0
    pltpu.sync_copy(acc_sp.at[pl.ds(row0, ROWS_PER_TILE)],
                    acc_out.at[pl.ds(out0, ROWS_PER_TILE)])


_R = 1024


def _prep_body(x_ref, ds_ref, dd_ref, g0_ref, wn_ref, ws_ref):
    deg_s = ds_ref[:, :1]
    deg_d = dd_ref[:, :1]
    dis = jnp.where(deg_s > 0, lax.rsqrt(jnp.maximum(deg_s, 1e-12)), 0.0)
    did = jnp.where(deg_d > 0, lax.rsqrt(jnp.maximum(deg_d, 1e-12)), 0.0)
    g0_ref[...] = x_ref[...] * dis
    wn_ref[...] = jnp.broadcast_to(dis * did, (_R, D))
    ws_ref[...] = jnp.broadcast_to(did, (_R, D))


def _tc_prep(x_pad, deg):
    spec_x = pl.BlockSpec((_R, D), lambda i: (i, jnp.int32(0)))
    spec_b = pl.BlockSpec((_R, D), lambda i: (i + jnp.int32(NPAD // _R), jnp.int32(0)))
    return pl.pallas_call(
        _prep_body,
        grid=(NPAD // _R,),
        in_specs=[spec_x, spec_x, spec_b],
        out_specs=[spec_x, spec_x, spec_x],
        out_shape=[jax.ShapeDtypeStruct((NPAD, D), jnp.float32)] * 3,
    )(x_pad, deg, deg)


def _node_body(last, acca_ref, accb_ref, wn_ref, ws_ref, sum_ref,
               *out_refs):
    acc = acca_ref[...] + accb_ref[...]
    h = acc * ws_ref[...]
    if last:
        out_refs[0][...] = (sum_ref[...] + h) * 0.25
    else:
        out_refs[0][...] = acc * wn_ref[...]
        out_refs[1][...] = sum_ref[...] + h


def _tc_node(acc_part, wn, ws, sum_in, last):
    spec = pl.BlockSpec((_R, D), lambda i: (i, jnp.int32(0)))
    spec_b = pl.BlockSpec((_R, D), lambda i: (i + jnp.int32(NPAD // _R), jnp.int32(0)))
    n_out = 1 if last else 2
    return pl.pallas_call(
        functools.partial(_node_body, last),
        grid=(NPAD // _R,),
        in_specs=[spec, spec_b, spec, spec, spec],
        out_specs=[spec] * n_out,
        out_shape=[jax.ShapeDtypeStruct((NPAD, D), jnp.float32)] * n_out,
    )(acc_part, acc_part, wn, ws, sum_in)


def kernel(x, edge_index):
    src = edge_index[0].astype(jnp.int32)
    dst = edge_index[1].astype(jnp.int32)
    pad = jnp.full((E_PAD - E,), PAD_NODE, jnp.int32)
    src_flat = jnp.concatenate([src, pad]).reshape(TOT_CHK, CH)
    dst_flat = jnp.concatenate([dst, pad]).reshape(TOT_CHK, CH)
    pada = jnp.full((NS, MAXN - NA, CH), PAD_NODE, jnp.int32)

    def blocked(flat):
        blk_a = flat[:NS * NA].reshape(NS, NA, CH)
        blk_b = flat[NS * NA:].reshape(NS, NB, CH)
        return jnp.concatenate([jnp.concatenate([blk_a, pada], 1), blk_b])

    src_blk = blocked(src_flat)
    dst_blk = blocked(dst_flat)
    x_pad = jnp.pad(x.astype(jnp.float32), ((0, NPAD - N_NODES), (0, 0)))
    onesD = jnp.ones((CH, D), jnp.float32)
    zD = jnp.zeros((NPAD, D), jnp.float32)

    idx_all = jnp.stack([src_flat, dst_flat])
    deg = _sc_degrees(idx_all, onesD, zD)
    g, wn, ws = _tc_prep(x_pad, deg)
    s = x_pad
    for layer in range(N_LAYERS):
        acc = _sc_scatter(g, src_blk, dst_blk, zD)
        if layer < N_LAYERS - 1:
            g, s = _tc_node(acc, wn, ws, s, last=False)
        else:
            (s,) = _tc_node(acc, wn, ws, s, last=True)
    out = s[:N_NODES]
    return out[:NUM_USERS], out[NUM_USERS:]


# speedup vs baseline: 1.7086x; 1.7086x over previous
import functools

import jax
import jax.numpy as jnp
from jax import lax
from jax.experimental import pallas as pl
from jax.experimental.pallas import tpu as pltpu
from jax.experimental.pallas import tpu_sc as plsc

N_NODES = 10000
NUM_USERS = 4000
D = 128
E = 320000
N_LAYERS = 3

NC = 2
NS = 16
NW = NC * NS
CH = 128
NCHK = 79
E_PAD = NW * NCHK * CH
NPAD = 10240
PAD_NODE = NPAD - 1
ROWS_PER_TILE = NPAD // NS

_mesh = plsc.VectorSubcoreMesh(core_axis_name="c", subcore_axis_name="s")


@functools.partial(
    pl.kernel,
    out_type=jax.ShapeDtypeStruct((NC * NPAD, D), jnp.float32),
    mesh=_mesh,
    scratch_types=[
        pltpu.VMEM((2, NCHK, CH), jnp.int32),
        pltpu.VMEM((CH, D), jnp.float32),
        pltpu.VMEM_SHARED((NPAD, D), jnp.float32),
    ],
)
def _sc_degrees(idx_hbm, ones_hbm, zD_hbm,
                deg_out,
                idx_v, ones_v, hist_sp):
    c = lax.axis_index("c").astype(jnp.int32)
    s = lax.axis_index("s").astype(jnp.int32)
    row0 = s * jnp.int32(ROWS_PER_TILE)
    pltpu.sync_copy(zD_hbm.at[pl.ds(row0, ROWS_PER_TILE)],
                    hist_sp.at[pl.ds(row0, ROWS_PER_TILE)])
    pltpu.sync_copy(ones_hbm, ones_v)
    pltpu.sync_copy(idx_hbm.at[c, pl.ds(s * jnp.int32(2), 2)], idx_v)
    plsc.subcore_barrier()

    @pl.loop(jnp.int32(0), jnp.int32(NCHK), step=jnp.int32(1))
    def _chunk(j):
        pltpu.sync_copy(ones_v, hist_sp.at[idx_v.at[jnp.int32(0), j]], add=True)
        pltpu.sync_copy(ones_v, hist_sp.at[idx_v.at[jnp.int32(1), j]], add=True)

    plsc.subcore_barrier()
    out0 = c * jnp.int32(NPAD) + row0
    pltpu.sync_copy(hist_sp.at[pl.ds(row0, ROWS_PER_TILE)],
                    deg_out.at[pl.ds(out0, ROWS_PER_TILE)])


@functools.partial(
    pl.kernel,
    out_type=jax.ShapeDtypeStruct((NC * NPAD, D), jnp.float32),
    mesh=_mesh,
    scratch_types=[
        pltpu.VMEM((NCHK, CH), jnp.int32),
        pltpu.VMEM((NCHK, CH), jnp.int32),
        pltpu.VMEM((CH, D), jnp.float32),
        pltpu.VMEM_SHARED((NPAD, D), jnp.float32),
        pltpu.SemaphoreType.DMA,
    ],
)
def _sc_scatter(g_hbm, src_hbm, dst_hbm, zD_hbm,
                acc_out,
                idx_s, idx_d, rows_v, acc_sp, sem):
    c = lax.axis_index("c").astype(jnp.int32)
    s = lax.axis_index("s").astype(jnp.int32)
    wid = c * jnp.int32(NS) + s
    row0 = s * jnp.int32(ROWS_PER_TILE)
    pltpu.sync_copy(zD_hbm.at[pl.ds(row0, ROWS_PER_TILE)],
                    acc_sp.at[pl.ds(row0, ROWS_PER_TILE)])
    pltpu.sync_copy(src_hbm.at[wid], idx_s)
    pltpu.sync_copy(dst_hbm.at[wid], idx_d)
    plsc.subcore_barrier()

    @pl.loop(jnp.int32(0), jnp.int32(NCHK), step=jnp.int32(1))
    def _chunk(j):
        pltpu.async_copy(g_hbm.at[idx_s.at[j]], rows_v, sem).wait()
        pltpu.sync_copy(rows_v, acc_sp.at[idx_d.at[j]], add=True)

    plsc.subcore_barrier()
    out0 = c * jnp.int32(NPAD) + row0
    pltpu.sync_copy(acc_sp.at[pl.ds(row0, ROWS_PER_TILE)],
                    acc_out.at[pl.ds(out0, ROWS_PER_TILE)])


_R = 1024


def _prep_body(x_ref, ds_ref, dd_ref, g0_ref, wn_ref, ws_ref):
    deg_s = ds_ref[:, :1]
    deg_d = dd_ref[:, :1]
    dis = jnp.where(deg_s > 0, lax.rsqrt(jnp.maximum(deg_s, 1e-12)), 0.0)
    did = jnp.where(deg_d > 0, lax.rsqrt(jnp.maximum(deg_d, 1e-12)), 0.0)
    g0_ref[...] = x_ref[...] * dis
    wn_ref[...] = jnp.broadcast_to(dis * did, (_R, D))
    ws_ref[...] = jnp.broadcast_to(did, (_R, D))


def _tc_prep(x_pad, deg):
    spec_x = pl.BlockSpec((_R, D), lambda i: (i, jnp.int32(0)))
    spec_b = pl.BlockSpec((_R, D), lambda i: (i + jnp.int32(NPAD // _R), jnp.int32(0)))
    return pl.pallas_call(
        _prep_body,
        grid=(NPAD // _R,),
        in_specs=[spec_x, spec_x, spec_b],
        out_specs=[spec_x, spec_x, spec_x],
        out_shape=[jax.ShapeDtypeStruct((NPAD, D), jnp.float32)] * 3,
    )(x_pad, deg, deg)


def _node_body(last, acca_ref, accb_ref, wn_ref, ws_ref, sum_ref,
               *out_refs):
    acc = acca_ref[...] + accb_ref[...]
    h = acc * ws_ref[...]
    if last:
        out_refs[0][...] = (sum_ref[...] + h) * 0.25
    else:
        out_refs[0][...] = acc * wn_ref[...]
        out_refs[1][...] = sum_ref[...] + h


def _tc_node(acc_part, wn, ws, sum_in, last):
    spec = pl.BlockSpec((_R, D), lambda i: (i, jnp.int32(0)))
    spec_b = pl.BlockSpec((_R, D), lambda i: (i + jnp.int32(NPAD // _R), jnp.int32(0)))
    n_out = 1 if last else 2
    return pl.pallas_call(
        functools.partial(_node_body, last),
        grid=(NPAD // _R,),
        in_specs=[spec, spec_b, spec, spec, spec],
        out_specs=[spec] * n_out,
        out_shape=[jax.ShapeDtypeStruct((NPAD, D), jnp.float32)] * n_out,
    )(acc_part, acc_part, wn, ws, sum_in)


def kernel(x, edge_index):
    src = edge_index[0].astype(jnp.int32)
    dst = edge_index[1].astype(jnp.int32)
    pad = jnp.full((E_PAD - E,), PAD_NODE, jnp.int32)
    src_blk = jnp.concatenate([src, pad]).reshape(NW, NCHK, CH)
    dst_blk = jnp.concatenate([dst, pad]).reshape(NW, NCHK, CH)
    x_pad = jnp.pad(x.astype(jnp.float32), ((0, NPAD - N_NODES), (0, 0)))
    onesD = jnp.ones((CH, D), jnp.float32)
    zD = jnp.zeros((NPAD, D), jnp.float32)

    idx_all = jnp.stack([src_blk, dst_blk])
    deg = _sc_degrees(idx_all, onesD, zD)
    g, wn, ws = _tc_prep(x_pad, deg)
    s = x_pad
    for layer in range(N_LAYERS):
        acc = _sc_scatter(g, src_blk, dst_blk, zD)
        if layer < N_LAYERS - 1:
            g, s = _tc_node(acc, wn, ws, s, last=False)
        else:
            (s,) = _tc_node(acc, wn, ws, s, last=True)
    out = s[:N_NODES]
    return out[:NUM_USERS], out[NUM_USERS:]
